# baseline (speedup 1.0000x reference)
import functools
import math

import jax
import jax.numpy as jnp
from jax import lax
from jax.experimental import pallas as pl
from jax.experimental.pallas import tpu as pltpu
from jax.experimental.pallas import tpu_sc as plsc

D_MODEL = 64
SEQ = 200
BATCH = 1024
N_ROWS = SEQ * BATCH
CHUNK = 128
CHUNKS_PER_SEQ = BATCH // CHUNK
N_CHUNKS = N_ROWS // CHUNK
NC, NS = 2, 16
NW = NC * NS
CPW = N_CHUNKS // NW
SCALE = math.sqrt(D_MODEL)
LANES = 16
GROUPS = D_MODEL // LANES
ROW_UNROLL = 4


def _body(idx_hbm, table_hbm, pe_hbm, out_hbm,
          idx_all, a0, a1, o0, o1, pe_v, g0, g1, s0, s1):
    w = lax.axis_index("s") * NC + lax.axis_index("c")
    c0 = w * CPW
    pltpu.sync_copy(pe_hbm, pe_v)
    pltpu.sync_copy(idx_hbm.at[w], idx_all)
    bufs = ((a0, o0, g0, s0), (a1, o1, g1, s1))

    for p in range(2):
        pltpu.async_copy(table_hbm.at[idx_all.at[p]], bufs[p][0], bufs[p][2])

    def round_body(r, carry):
        for p in range(2):
            a, o, gsem, osem = bufs[p]
            k = 2 * r + p
            c = c0 + k
            pltpu.make_async_copy(table_hbm.at[idx_all.at[k]], a, gsem).wait()

            @pl.when(r >= 1)
            def _wait_prev_scatter():
                pltpu.make_async_copy(
                    o, out_hbm.at[pl.ds((c - 2) * CHUNK, CHUNK)], osem).wait()

            s_off = (c // CHUNKS_PER_SEQ) * D_MODEL
            pe_vecs = [pe_v[pl.ds(s_off + LANES * j, LANES)]
                       for j in range(GROUPS)]

            def row_body(i, _):
                for u in range(ROW_UNROLL):
                    row = i * ROW_UNROLL + u
                    for j in range(GROUPS):
                        o[row, pl.ds(LANES * j, LANES)] = (
                            a[row, pl.ds(LANES * j, LANES)] * SCALE
                            + pe_vecs[j])
                return 0

            lax.fori_loop(0, CHUNK // ROW_UNROLL, row_body, 0)
            pltpu.async_copy(o, out_hbm.at[pl.ds(c * CHUNK, CHUNK)], osem)

            @pl.when(k + 2 < CPW)
            def _issue_next_gather():
                pltpu.async_copy(
                    table_hbm.at[idx_all.at[k + 2]], a, gsem)
        return carry

    lax.fori_loop(0, CPW // 2, round_body, 0)

    for p in range(2):
        c = c0 + CPW - 2 + p
        pltpu.make_async_copy(
            bufs[p][1], out_hbm.at[pl.ds(c * CHUNK, CHUNK)], bufs[p][3]).wait()


@jax.jit
def _emb_pe(idx2d, table, pe_flat):
    mesh = plsc.VectorSubcoreMesh(core_axis_name="c", subcore_axis_name="s")
    return pl.kernel(
        _body,
        out_type=jax.ShapeDtypeStruct((N_ROWS, D_MODEL), jnp.float32),
        mesh=mesh,
        compiler_params=pltpu.CompilerParams(use_tc_tiling_on_sc=False),
        scratch_types=[
            pltpu.VMEM((CPW, CHUNK), jnp.int32),
            pltpu.VMEM((CHUNK, D_MODEL), jnp.float32),
            pltpu.VMEM((CHUNK, D_MODEL), jnp.float32),
            pltpu.VMEM((CHUNK, D_MODEL), jnp.float32),
            pltpu.VMEM((CHUNK, D_MODEL), jnp.float32),
            pltpu.VMEM((SEQ * D_MODEL,), jnp.float32),
            pltpu.SemaphoreType.DMA,
            pltpu.SemaphoreType.DMA,
            pltpu.SemaphoreType.DMA,
            pltpu.SemaphoreType.DMA,
        ],
    )(idx2d, table, pe_flat)


def kernel(sparse_input, table, pe):
    seq, batch = sparse_input.shape
    idx2d = sparse_input.astype(jnp.int32).reshape(NW, CPW, CHUNK)
    pe_flat = pe[:seq].reshape(seq * D_MODEL)
    out = _emb_pe(idx2d, table, pe_flat)
    return out.reshape(seq, batch, D_MODEL)
